# Initial kernel scaffold; baseline (speedup 1.0000x reference)
import jax
import jax.numpy as jnp
from jax.experimental import pallas as pl


def kernel(node_features, edge_dst, edge_src, W_q, W_k, W_v, W_dot):
    raise NotImplementedError("write your pallas kernel here")


# speedup vs baseline: 4.3080x; 4.3080x over previous
import functools

import jax
import jax.numpy as jnp
from jax import lax
from jax.experimental import pallas as pl
from jax.experimental.pallas import tpu as pltpu
from jax.experimental.pallas import tpu_sc as plsc

N = 10000
E = 320000
D = 128
H = 4

NC = 2
NS = 16
L = 16
NW = NC * NS
EPW = E // NW
C = 80
NCHUNK = EPW // C
GPC = C // L

_EPS = 1e-5


def _na(x):
    n = jnp.abs(x)
    return x * jax.nn.sigmoid(n) / (n + _EPS)


def _pre_body(f_ref, wq_ref, wk_ref, wv_ref, wd2_ref, qw_ref, kn_ref, vn_ref):
    x = f_ref[...]
    q = _na(jnp.dot(x, wq_ref[...], preferred_element_type=jnp.float32))
    qw_ref[...] = jnp.dot(q, wd2_ref[...], preferred_element_type=jnp.float32)
    kn_ref[...] = _na(jnp.dot(x, wk_ref[...], preferred_element_type=jnp.float32))
    vn_ref[...] = _na(jnp.dot(x, wv_ref[...], preferred_element_type=jnp.float32))


def _tc_pre(f, w_q, w_k, w_v, wd2):
    rb = 1000
    grid = (N // rb,)
    return pl.pallas_call(
        _pre_body,
        grid=grid,
        in_specs=[
            pl.BlockSpec((rb, D), lambda i: (i, 0)),
            pl.BlockSpec((D, D), lambda i: (0, 0)),
            pl.BlockSpec((D, D), lambda i: (0, 0)),
            pl.BlockSpec((D, D), lambda i: (0, 0)),
            pl.BlockSpec((D, H * D), lambda i: (0, 0)),
        ],
        out_specs=[
            pl.BlockSpec((rb, H * D), lambda i: (i, 0)),
            pl.BlockSpec((rb, D), lambda i: (i, 0)),
            pl.BlockSpec((rb, D), lambda i: (i, 0)),
        ],
        out_shape=[
            jax.ShapeDtypeStruct((N, H * D), jnp.float32),
            jax.ShapeDtypeStruct((N, D), jnp.float32),
            jax.ShapeDtypeStruct((N, D), jnp.float32),
        ],
    )(f, w_q, w_k, w_v, wd2)


def _pass1_body(qw_hbm, kn_hbm, dst_hbm, src_hbm, expv_hbm, zall_hbm,
                dst_v, src_v, qw_v, kn_v, ev_v, z_v, stage_v, sem1, sem2):
    cid = lax.axis_index("c")
    sid = lax.axis_index("s")
    wid = sid * NC + cid
    base0 = wid * EPW
    lane = lax.iota(jnp.int32, L)
    zero16 = jnp.zeros((L,), jnp.float32)

    def _zi(i, _):
        z_v[pl.ds(i * L, L)] = zero16
        return 0
    lax.fori_loop(0, N // L, _zi, 0)

    def chunk_body(i, _):
        base = base0 + i * C
        pltpu.sync_copy(dst_hbm.at[pl.ds(base, C)], dst_v)
        pltpu.sync_copy(src_hbm.at[pl.ds(base, C)], src_v)
        cp1 = pltpu.async_copy(qw_hbm.at[dst_v], qw_v, sem1)
        cp2 = pltpu.async_copy(kn_hbm.at[src_v], kn_v, sem2)
        cp1.wait()
        cp2.wait()

        def group_body(g, _):
            def edge_body(l, _):
                r = g * L + l
                kb = [kn_v[r, pl.ds(b * L, L)] for b in range(D // L)]
                for h in range(H):
                    acc = qw_v[r, pl.ds(h * D, L)] * kb[0]
                    for b in range(1, D // L):
                        acc = acc + qw_v[r, pl.ds(h * D + b * L, L)] * kb[b]
                    stage_v[pl.ds((h * L + l) * L, L)] = acc
                return 0

            lax.fori_loop(0, L, edge_body, 0)
            ev = zero16
            for h in range(H):
                row = (h * L + lane) * L
                d = plsc.load_gather(stage_v, [row])
                for p in range(1, L):
                    d = d + plsc.load_gather(stage_v, [row + p])
                ev = ev + jnp.exp(d)
            ev = ev * 0.25
            ev_v[pl.ds(g * L, L)] = ev
            dst16 = dst_v[pl.ds(g * L, L)]
            plsc.addupdate_scatter(z_v, [dst16], ev)
            return 0

        lax.fori_loop(0, GPC, group_body, 0)
        pltpu.sync_copy(ev_v, expv_hbm.at[pl.ds(base, C)])
        return 0

    lax.fori_loop(0, NCHUNK, chunk_body, 0)
    pltpu.sync_copy(z_v, zall_hbm.at[pl.ds(wid * N, N)])


def _sc_pass1(qw, kn, dst, src):
    mesh = plsc.VectorSubcoreMesh(core_axis_name="c", subcore_axis_name="s",
                                  num_cores=NC, num_subcores=NS)
    return pl.kernel(
        _pass1_body,
        compiler_params=pltpu.CompilerParams(needs_layout_passes=False),
        out_type=[
            jax.ShapeDtypeStruct((E,), jnp.float32),
            jax.ShapeDtypeStruct((NW * N,), jnp.float32),
        ],
        mesh=mesh,
        scratch_types=[
            pltpu.VMEM((C,), jnp.int32),
            pltpu.VMEM((C,), jnp.int32),
            pltpu.VMEM((C, H * D), jnp.float32),
            pltpu.VMEM((C, D), jnp.float32),
            pltpu.VMEM((C,), jnp.float32),
            pltpu.VMEM((N,), jnp.float32),
            pltpu.VMEM((H * L * L,), jnp.float32),
            pltpu.SemaphoreType.DMA,
            pltpu.SemaphoreType.DMA,
        ],
    )(qw, kn, dst, src)


def _mid_body(zall_ref, ev_ref, szinv_ref, sexpv_ref):
    z = jnp.sum(zall_ref[...], axis=0)
    z = jnp.where(z == 0.0, 1.0, z)
    szinv_ref[...] = lax.rsqrt(z)
    sexpv_ref[...] = jnp.sqrt(ev_ref[...])


def _tc_mid(zall, expv):
    za = zall.reshape(NW, 80, 125)
    ev = expv.reshape(2500, 128)
    szinv, sexpv = pl.pallas_call(
        _mid_body,
        out_shape=[
            jax.ShapeDtypeStruct((80, 125), jnp.float32),
            jax.ShapeDtypeStruct((2500, 128), jnp.float32),
        ],
    )(za, ev)
    return szinv.reshape(N), sexpv.reshape(E)


def _pass2_body(vn_hbm, dst_hbm, src_hbm, se_hbm, szi_hbm, fout_hbm,
                dst_v, src_v, vr_v, w_v, se_v, szi_v, fout_sh, sem1):
    cid = lax.axis_index("c")
    sid = lax.axis_index("s")
    base0 = (sid * NC + cid) * EPW

    pltpu.sync_copy(szi_hbm, szi_v)

    zero16 = jnp.zeros((L,), jnp.float32)

    def _zr(i, _):
        vr_v[i, pl.ds(0, L)] = zero16
        for b in range(1, D // L):
            vr_v[i, pl.ds(b * L, L)] = zero16
        return 0
    lax.fori_loop(0, C, _zr, 0)

    nchips = N // C

    def _zs(t, _):
        j = sid + t * NS

        @pl.when(j < nchips)
        def _():
            row0 = pl.multiple_of(j * C, 8)
            pltpu.sync_copy(vr_v, fout_sh.at[pl.ds(row0, C)])
        return 0
    lax.fori_loop(0, (nchips + NS - 1) // NS, _zs, 0)

    plsc.subcore_barrier()

    def chunk_body(i, _):
        base = base0 + i * C
        pltpu.sync_copy(dst_hbm.at[pl.ds(base, C)], dst_v)
        pltpu.sync_copy(src_hbm.at[pl.ds(base, C)], src_v)
        pltpu.sync_copy(se_hbm.at[pl.ds(base, C)], se_v)
        pltpu.async_copy(vn_hbm.at[src_v], vr_v, sem1).wait()

        def group_body(g, _):
            dst16 = dst_v[pl.ds(g * L, L)]
            se16 = se_v[pl.ds(g * L, L)]
            szg = plsc.load_gather(szi_v, [dst16])
            w_v[pl.ds(g * L, L)] = se16 * szg
            return 0
        lax.fori_loop(0, GPC, group_body, 0)

        def row_body(r, _):
            wspl = plsc.load_gather(w_v, [jnp.full((L,), r, jnp.int32)])
            for b in range(D // L):
                vr_v[r, pl.ds(b * L, L)] = vr_v[r, pl.ds(b * L, L)] * wspl
            return 0
        lax.fori_loop(0, C, row_body, 0)

        pltpu.sync_copy(vr_v, fout_sh.at[dst_v], add=True)
        return 0

    lax.fori_loop(0, NCHUNK, chunk_body, 0)
    plsc.subcore_barrier()

    def _wb(t, _):
        j = sid + t * NS

        @pl.when(j < nchips)
        def _():
            row0 = pl.multiple_of(j * C, 8)
            pltpu.sync_copy(fout_sh.at[pl.ds(row0, C)],
                            fout_hbm.at[cid, pl.ds(row0, C)])
        return 0
    lax.fori_loop(0, (nchips + NS - 1) // NS, _wb, 0)


def _sc_pass2(vn, dst, src, sexpv, szinv):
    mesh = plsc.VectorSubcoreMesh(core_axis_name="c", subcore_axis_name="s",
                                  num_cores=NC, num_subcores=NS)
    return pl.kernel(
        _pass2_body,
        compiler_params=pltpu.CompilerParams(needs_layout_passes=False),
        out_type=jax.ShapeDtypeStruct((NC, N, D), jnp.float32),
        mesh=mesh,
        scratch_types=[
            pltpu.VMEM((C,), jnp.int32),
            pltpu.VMEM((C,), jnp.int32),
            pltpu.VMEM((C, D), jnp.float32),
            pltpu.VMEM((C,), jnp.float32),
            pltpu.VMEM((C,), jnp.float32),
            pltpu.VMEM((N,), jnp.float32),
            pltpu.VMEM_SHARED((N, D), jnp.float32),
            pltpu.SemaphoreType.DMA,
        ],
    )(vn, dst, src, sexpv, szinv)


def _post_body(fp_ref, vn_ref, out_ref):
    out_ref[...] = fp_ref[0] + fp_ref[1] + vn_ref[...]


def _tc_post(fout, vn):
    rb = 1000
    return pl.pallas_call(
        _post_body,
        grid=(N // rb,),
        in_specs=[
            pl.BlockSpec((NC, rb, D), lambda i: (0, i, 0)),
            pl.BlockSpec((rb, D), lambda i: (i, 0)),
        ],
        out_specs=pl.BlockSpec((rb, D), lambda i: (i, 0)),
        out_shape=jax.ShapeDtypeStruct((N, D), jnp.float32),
    )(fout, vn)


@jax.jit
def kernel(node_features, edge_dst, edge_src, W_q, W_k, W_v, W_dot):
    dst = edge_dst.astype(jnp.int32)
    src = edge_src.astype(jnp.int32)
    wd2 = jnp.transpose(W_dot, (1, 0, 2)).reshape(D, H * D)

    qw, kn, vn = _tc_pre(node_features, W_q, W_k, W_v, wd2)
    expv, zall = _sc_pass1(qw, kn, dst, src)
    szinv, sexpv = _tc_mid(zall, expv)
    fout = _sc_pass2(vn, dst, src, sexpv, szinv)
    return _tc_post(fout, vn)


# speedup vs baseline: 6.0176x; 1.3969x over previous
import functools

import jax
import jax.numpy as jnp
from jax import lax
from jax.experimental import pallas as pl
from jax.experimental.pallas import tpu as pltpu
from jax.experimental.pallas import tpu_sc as plsc

N = 10000
E = 320000
D = 128
H = 4

NC = 2
NS = 16
L = 16
NW = NC * NS
EPW = E // NW
C = 64
NFULL = EPW // C
PAIRS = NFULL // 2
TAIL = EPW - NFULL * C
TBASE = NFULL * C
GPC = C // L
DB = D // L

_EPS = 1e-5


def _na(x):
    n = jnp.abs(x)
    return x * jax.nn.sigmoid(n) / (n + _EPS)


def _pre_body(f_ref, wq_ref, wk_ref, wv_ref, wd2_ref, qw_ref, kn_ref, vn_ref):
    x = f_ref[...]
    q = _na(jnp.dot(x, wq_ref[...], preferred_element_type=jnp.float32))
    qw_ref[...] = jnp.dot(q, wd2_ref[...], preferred_element_type=jnp.float32)
    kn_ref[...] = _na(jnp.dot(x, wk_ref[...], preferred_element_type=jnp.float32))
    vn_ref[...] = _na(jnp.dot(x, wv_ref[...], preferred_element_type=jnp.float32))


def _tc_pre(f, w_q, w_k, w_v, wd2):
    rb = 1000
    return pl.pallas_call(
        _pre_body,
        grid=(N // rb,),
        in_specs=[
            pl.BlockSpec((rb, D), lambda i: (i, 0)),
            pl.BlockSpec((D, D), lambda i: (0, 0)),
            pl.BlockSpec((D, D), lambda i: (0, 0)),
            pl.BlockSpec((D, D), lambda i: (0, 0)),
            pl.BlockSpec((D, H * D), lambda i: (0, 0)),
        ],
        out_specs=[
            pl.BlockSpec((rb, H * D), lambda i: (i, 0)),
            pl.BlockSpec((rb, D), lambda i: (i, 0)),
            pl.BlockSpec((rb, D), lambda i: (i, 0)),
        ],
        out_shape=[
            jax.ShapeDtypeStruct((N, H * D), jnp.float32),
            jax.ShapeDtypeStruct((N, D), jnp.float32),
            jax.ShapeDtypeStruct((N, D), jnp.float32),
        ],
    )(f, w_q, w_k, w_v, wd2)


def _pass1_body(qw_hbm, kn_hbm, dst_hbm, src_hbm, expv_hbm, zall_hbm,
                dst_v, src_v, qw_a, qw_b, kn_a, kn_b, ev_v, z_v, stage_v,
                sem_a, sem_b):
    cid = lax.axis_index("c")
    sid = lax.axis_index("s")
    wid = sid * NC + cid
    base0 = wid * EPW
    lane = lax.iota(jnp.int32, L)
    zero16 = jnp.zeros((L,), jnp.float32)

    cpd = pltpu.async_copy(dst_hbm.at[pl.ds(base0, EPW)], dst_v, sem_a)
    cps = pltpu.async_copy(src_hbm.at[pl.ds(base0, EPW)], src_v, sem_a)

    def _zi(i, _):
        z_v[pl.ds(i * L, L)] = zero16
        return 0
    lax.fori_loop(0, N // L, _zi, 0)
    cpd.wait()
    cps.wait()

    def fire(i, qwb, knb, sem):
        pltpu.async_copy(qw_hbm.at[dst_v.at[pl.ds(i * C, C)]], qwb, sem)
        pltpu.async_copy(kn_hbm.at[src_v.at[pl.ds(i * C, C)]], knb, sem)

    def waitg(qwb, knb, sem):
        pltpu.make_async_copy(qw_hbm.at[pl.ds(0, C)], qwb, sem).wait()
        pltpu.make_async_copy(kn_hbm.at[pl.ds(0, C)], knb, sem).wait()

    def group_compute(qwb, knb, rbase, ebase):
        def edge_body(l, _):
            r = rbase + l
            kb = [knb[r, pl.ds(b * L, L)] for b in range(DB)]
            for h in range(H):
                acc = qwb[r, pl.ds(h * D, L)] * kb[0]
                for b in range(1, DB):
                    acc = acc + qwb[r, pl.ds(h * D + b * L, L)] * kb[b]
                stage_v[pl.ds((h * L + l) * L, L)] = acc
            return 0

        lax.fori_loop(0, L, edge_body, 0)
        ev = zero16
        for h in range(H):
            row = (h * L + lane) * L
            d = plsc.load_gather(stage_v, [row])
            for p in range(1, L):
                d = d + plsc.load_gather(stage_v, [row + p])
            ev = ev + jnp.exp(d)
        ev = ev * 0.25
        ev_v[pl.ds(ebase, L)] = ev
        dst16 = dst_v[pl.ds(ebase, L)]
        plsc.addupdate_scatter(z_v, [dst16], ev)

    def compute(i, qwb, knb):
        def g_body(g, _):
            group_compute(qwb, knb, g * L, i * C + g * L)
            return 0
        lax.fori_loop(0, GPC, g_body, 0)

    fire(0, qw_a, kn_a, sem_a)

    def pair_body(t, _):
        i = 2 * t
        fire(i + 1, qw_b, kn_b, sem_b)
        waitg(qw_a, kn_a, sem_a)
        compute(i, qw_a, kn_a)

        @pl.when(t < PAIRS - 1)
        def _():
            fire(i + 2, qw_a, kn_a, sem_a)

        waitg(qw_b, kn_b, sem_b)
        compute(i + 1, qw_b, kn_b)
        return 0

    lax.fori_loop(0, PAIRS, pair_body, 0)

    tq = pltpu.async_copy(qw_hbm.at[dst_v.at[pl.ds(TBASE, TAIL)]],
                          qw_a.at[pl.ds(0, TAIL)], sem_a)
    tk = pltpu.async_copy(kn_hbm.at[src_v.at[pl.ds(TBASE, TAIL)]],
                          kn_a.at[pl.ds(0, TAIL)], sem_a)
    tq.wait()
    tk.wait()
    group_compute(qw_a, kn_a, 0, TBASE)

    pltpu.sync_copy(ev_v, expv_hbm.at[pl.ds(base0, EPW)])
    pltpu.sync_copy(z_v, zall_hbm.at[pl.ds(wid * N, N)])


def _sc_pass1(qw, kn, dst, src):
    mesh = plsc.VectorSubcoreMesh(core_axis_name="c", subcore_axis_name="s",
                                  num_cores=NC, num_subcores=NS)
    return pl.kernel(
        _pass1_body,
        compiler_params=pltpu.CompilerParams(needs_layout_passes=False),
        out_type=[
            jax.ShapeDtypeStruct((E,), jnp.float32),
            jax.ShapeDtypeStruct((NW * N,), jnp.float32),
        ],
        mesh=mesh,
        scratch_types=[
            pltpu.VMEM((EPW,), jnp.int32),
            pltpu.VMEM((EPW,), jnp.int32),
            pltpu.VMEM((C, H * D), jnp.float32),
            pltpu.VMEM((C, H * D), jnp.float32),
            pltpu.VMEM((C, D), jnp.float32),
            pltpu.VMEM((C, D), jnp.float32),
            pltpu.VMEM((EPW,), jnp.float32),
            pltpu.VMEM((N,), jnp.float32),
            pltpu.VMEM((H * L * L,), jnp.float32),
            pltpu.SemaphoreType.DMA,
            pltpu.SemaphoreType.DMA,
        ],
    )(qw, kn, dst, src)


def _mid_body(zall_ref, ev_ref, szinv_ref, sexpv_ref):
    z = jnp.sum(zall_ref[...], axis=0)
    z = jnp.where(z == 0.0, 1.0, z)
    szinv_ref[...] = lax.rsqrt(z)
    sexpv_ref[...] = jnp.sqrt(ev_ref[...])


def _tc_mid(zall, expv):
    za = zall.reshape(NW, 80, 125)
    ev = expv.reshape(2500, 128)
    szinv, sexpv = pl.pallas_call(
        _mid_body,
        out_shape=[
            jax.ShapeDtypeStruct((80, 125), jnp.float32),
            jax.ShapeDtypeStruct((2500, 128), jnp.float32),
        ],
    )(za, ev)
    return szinv.reshape(N), sexpv.reshape(E)


C2 = 80
NCH2 = EPW // C2
G2 = C2 // L


def _pass2_body(vn_hbm, dst_hbm, src_hbm, se_hbm, szi_hbm, fout_hbm,
                dc_a, dc_b, sc_a, sc_b, se_a, se_b, szg_a, szg_b,
                vr_a, vr_b, sr_a, sr_b, ds_a, ds_b, w_v, fout_sh,
                sem_ia, sem_ib, sem_ga, sem_gb, sem_sa, sem_sb):
    cid = lax.axis_index("c")
    sid = lax.axis_index("s")
    wid = sid * NC + cid
    base0 = wid * EPW
    zero16 = jnp.zeros((L,), jnp.float32)

    def _zr(i, _):
        for b in range(DB):
            sr_a[i, pl.ds(b * L, L)] = zero16
        return 0
    lax.fori_loop(0, C2, _zr, 0)

    def _zs(t, _):
        j = sid + t * NS

        @pl.when(j < NCH2)
        def _():
            pltpu.sync_copy(sr_a, fout_sh.at[pl.ds(pl.multiple_of(j * C2, 8), C2)])
        return 0
    lax.fori_loop(0, (NCH2 + NS - 1) // NS, _zs, 0)
    plsc.subcore_barrier()

    def fire_idx(j, dc, scb, seb, sem):
        pltpu.async_copy(dst_hbm.at[pl.ds(base0 + j * C2, C2)], dc, sem)
        pltpu.async_copy(src_hbm.at[pl.ds(base0 + j * C2, C2)], scb, sem)
        pltpu.async_copy(se_hbm.at[pl.ds(base0 + j * C2, C2)], seb, sem)

    def wait_idx(dc, scb, seb, sem):
        pltpu.make_async_copy(dst_hbm.at[pl.ds(0, C2)], dc, sem).wait()
        pltpu.make_async_copy(src_hbm.at[pl.ds(0, C2)], scb, sem).wait()
        pltpu.make_async_copy(se_hbm.at[pl.ds(0, C2)], seb, sem).wait()

    def fire_g(dc, scb, vrb, szgb, sem):
        pltpu.async_copy(vn_hbm.at[scb], vrb, sem)
        pltpu.async_copy(szi_hbm.at[dc], szgb, sem)

    def wait_g(vrb, szgb, sem):
        pltpu.make_async_copy(vn_hbm.at[pl.ds(0, C2)], vrb, sem).wait()
        pltpu.make_async_copy(se_hbm.at[pl.ds(0, C2)], szgb, sem).wait()

    def fire_s(srb, dsb, sem):
        pltpu.async_copy(srb, fout_sh.at[dsb], sem, add=True)

    def wait_s(srb, sem):
        pltpu.make_async_copy(vn_hbm.at[pl.ds(0, C2)], srb, sem).wait()

    def compute(dc, seb, szgb, vrb, srb, dsb):
        def g_body(g, _):
            off = g * L
            w_v[pl.ds(off, L)] = seb[pl.ds(off, L)] * szgb[pl.ds(off, L)]
            dsb[pl.ds(off, L)] = dc[pl.ds(off, L)]
            return 0
        lax.fori_loop(0, G2, g_body, 0)

        def row_body(r, _):
            wspl = plsc.load_gather(w_v, [jnp.full((L,), r, jnp.int32)])
            for b in range(DB):
                srb[r, pl.ds(b * L, L)] = vrb[r, pl.ds(b * L, L)] * wspl
            return 0
        lax.fori_loop(0, C2, row_body, 0)

    bufs = ((dc_a, sc_a, se_a, szg_a, vr_a, sr_a, ds_a, sem_ia, sem_ga, sem_sa),
            (dc_b, sc_b, se_b, szg_b, vr_b, sr_b, ds_b, sem_ib, sem_gb, sem_sb))

    def phase(j, t, par):
        dc, scb, seb, szgb, vrb, srb, dsb, sem_i, sem_g, sem_s = bufs[par]
        dcq, scq, seq, szgq, vrq, srq, dsq, sem_iq, sem_gq, sem_sq = bufs[1 - par]
        @pl.when(j + 1 < NCH2)
        def _():
            wait_idx(dcq, scq, seq, sem_iq)
            fire_g(dcq, scq, vrq, szgq, sem_gq)
        wait_g(vrb, szgb, sem_g)

        @pl.when(j >= 2)
        def _():
            wait_s(srb, sem_s)

        compute(dc, seb, szgb, vrb, srb, dsb)
        fire_s(srb, dsb, sem_s)

        @pl.when(j + 2 < NCH2)
        def _():
            fire_idx(j + 2, dc, scb, seb, sem_i)

    fire_idx(0, dc_a, sc_a, se_a, sem_ia)
    fire_idx(1, dc_b, sc_b, se_b, sem_ib)
    wait_idx(dc_a, sc_a, se_a, sem_ia)
    fire_g(dc_a, sc_a, vr_a, szg_a, sem_ga)

    def pair_body(t, _):
        j = 2 * t
        phase(j, t, 0)
        phase(j + 1, t, 1)
        return 0
    lax.fori_loop(0, NCH2 // 2, pair_body, 0)
    phase(NCH2 - 1, NCH2 // 2, 0)

    wait_s(sr_a, sem_sa)
    wait_s(sr_b, sem_sb)
    plsc.subcore_barrier()

    def _wb(t, _):
        j = sid + t * NS

        @pl.when(j < NCH2)
        def _():
            row0 = pl.multiple_of(j * C2, 8)
            pltpu.sync_copy(fout_sh.at[pl.ds(row0, C2)],
                            fout_hbm.at[cid, pl.ds(row0, C2)])
        return 0
    lax.fori_loop(0, (NCH2 + NS - 1) // NS, _wb, 0)


def _sc_pass2(vn, dst, src, sexpv, szinv):
    mesh = plsc.VectorSubcoreMesh(core_axis_name="c", subcore_axis_name="s",
                                  num_cores=NC, num_subcores=NS)
    return pl.kernel(
        _pass2_body,
        compiler_params=pltpu.CompilerParams(needs_layout_passes=False),
        out_type=jax.ShapeDtypeStruct((NC, N, D), jnp.float32),
        mesh=mesh,
        scratch_types=[
            pltpu.VMEM((C2,), jnp.int32),
            pltpu.VMEM((C2,), jnp.int32),
            pltpu.VMEM((C2,), jnp.int32),
            pltpu.VMEM((C2,), jnp.int32),
            pltpu.VMEM((C2,), jnp.float32),
            pltpu.VMEM((C2,), jnp.float32),
            pltpu.VMEM((C2,), jnp.float32),
            pltpu.VMEM((C2,), jnp.float32),
            pltpu.VMEM((C2, D), jnp.float32),
            pltpu.VMEM((C2, D), jnp.float32),
            pltpu.VMEM((C2, D), jnp.float32),
            pltpu.VMEM((C2, D), jnp.float32),
            pltpu.VMEM((C2,), jnp.int32),
            pltpu.VMEM((C2,), jnp.int32),
            pltpu.VMEM((C2,), jnp.float32),
            pltpu.VMEM_SHARED((N, D), jnp.float32),
            pltpu.SemaphoreType.DMA,
            pltpu.SemaphoreType.DMA,
            pltpu.SemaphoreType.DMA,
            pltpu.SemaphoreType.DMA,
            pltpu.SemaphoreType.DMA,
            pltpu.SemaphoreType.DMA,
        ],
    )(vn, dst, src, sexpv, szinv)


def _post_body(fp_ref, vn_ref, out_ref):
    out_ref[...] = fp_ref[0] + fp_ref[1] + vn_ref[...]


def _tc_post(fout, vn):
    rb = 1000
    return pl.pallas_call(
        _post_body,
        grid=(N // rb,),
        in_specs=[
            pl.BlockSpec((NC, rb, D), lambda i: (0, i, 0)),
            pl.BlockSpec((rb, D), lambda i: (i, 0)),
        ],
        out_specs=pl.BlockSpec((rb, D), lambda i: (i, 0)),
        out_shape=jax.ShapeDtypeStruct((N, D), jnp.float32),
    )(fout, vn)


@jax.jit
def kernel(node_features, edge_dst, edge_src, W_q, W_k, W_v, W_dot):
    dst = edge_dst.astype(jnp.int32)
    src = edge_src.astype(jnp.int32)
    wd2 = jnp.transpose(W_dot, (1, 0, 2)).reshape(D, H * D)

    qw, kn, vn = _tc_pre(node_features, W_q, W_k, W_v, wd2)
    expv, zall = _sc_pass1(qw, kn, dst, src)
    szinv, sexpv = _tc_mid(zall, expv)
    fout = _sc_pass2(vn, dst, src, sexpv, szinv)
    return _tc_post(fout, vn)


# speedup vs baseline: 6.1267x; 1.0181x over previous
import functools

import jax
import jax.numpy as jnp
from jax import lax
from jax.experimental import pallas as pl
from jax.experimental.pallas import tpu as pltpu
from jax.experimental.pallas import tpu_sc as plsc

N = 10000
E = 320000
D = 128
H = 4

NC = 2
NS = 16
L = 16
NW = NC * NS
EPW = E // NW
C = 64
NFULL = EPW // C
PAIRS = NFULL // 2
TAIL = EPW - NFULL * C
TBASE = NFULL * C
GPC = C // L
DB = D // L

_EPS = 1e-5


def _na(x):
    n = jnp.abs(x)
    return x * jax.nn.sigmoid(n) / (n + _EPS)


def _pre_body(f_ref, wq_ref, wk_ref, wv_ref, wd2_ref, qw_ref, kn_ref, vn_ref):
    x = f_ref[...]
    q = _na(jnp.dot(x, wq_ref[...], preferred_element_type=jnp.float32))
    qw_ref[...] = jnp.dot(q, wd2_ref[...], preferred_element_type=jnp.float32)
    kn_ref[...] = _na(jnp.dot(x, wk_ref[...], preferred_element_type=jnp.float32))
    vn_ref[...] = _na(jnp.dot(x, wv_ref[...], preferred_element_type=jnp.float32))


def _tc_pre(f, w_q, w_k, w_v, wd2):
    rb = 1000
    return pl.pallas_call(
        _pre_body,
        grid=(N // rb,),
        in_specs=[
            pl.BlockSpec((rb, D), lambda i: (i, 0)),
            pl.BlockSpec((D, D), lambda i: (0, 0)),
            pl.BlockSpec((D, D), lambda i: (0, 0)),
            pl.BlockSpec((D, D), lambda i: (0, 0)),
            pl.BlockSpec((D, H * D), lambda i: (0, 0)),
        ],
        out_specs=[
            pl.BlockSpec((rb, H * D), lambda i: (i, 0)),
            pl.BlockSpec((rb, D), lambda i: (i, 0)),
            pl.BlockSpec((rb, D), lambda i: (i, 0)),
        ],
        out_shape=[
            jax.ShapeDtypeStruct((N, H * D), jnp.float32),
            jax.ShapeDtypeStruct((N, D), jnp.float32),
            jax.ShapeDtypeStruct((N, D), jnp.float32),
        ],
    )(f, w_q, w_k, w_v, wd2)


def _pack_pairs(x):
    n, m = x.shape
    xb = x.astype(jnp.bfloat16).reshape(n, m // 32, 2, L)
    xb = jnp.transpose(xb, (0, 1, 3, 2))
    return lax.bitcast_convert_type(xb, jnp.float32).reshape(n, m // 2)


C1 = 96
NF1 = EPW // C1
PAIRS1 = NF1 // 2
T1 = EPW - NF1 * C1
TB1 = NF1 * C1
G1 = C1 // L
QWW = H * D // 2
HW = D // 2
PB = HW // L


def _unpack2(w):
    return plsc.unpack(plsc.bitcast(w, jnp.bfloat16),
                       format=plsc.PackFormat.INTERLEAVED)


def _pass1_body(qw_hbm, kn_hbm, dst_hbm, src_hbm, expv_hbm, zall_hbm,
                dst_v, src_v, qw_a, qw_b, kn_a, kn_b, ev_v, z_v, stage_v,
                sem_a, sem_b):
    cid = lax.axis_index("c")
    sid = lax.axis_index("s")
    wid = sid * NC + cid
    base0 = wid * EPW
    lane = lax.iota(jnp.int32, L)
    zero16 = jnp.zeros((L,), jnp.float32)

    cpd = pltpu.async_copy(dst_hbm.at[pl.ds(base0, EPW)], dst_v, sem_a)
    cps = pltpu.async_copy(src_hbm.at[pl.ds(base0, EPW)], src_v, sem_a)

    def _zi(i, _):
        z_v[pl.ds(i * L, L)] = zero16
        return 0
    lax.fori_loop(0, N // L, _zi, 0)
    cpd.wait()
    cps.wait()

    def fire(i, qwb, knb, sem):
        pltpu.async_copy(qw_hbm.at[dst_v.at[pl.ds(i * C1, C1)]], qwb, sem)
        pltpu.async_copy(kn_hbm.at[src_v.at[pl.ds(i * C1, C1)]], knb, sem)

    def waitg(qwb, knb, sem):
        pltpu.make_async_copy(qw_hbm.at[pl.ds(0, C1)], qwb, sem).wait()
        pltpu.make_async_copy(kn_hbm.at[pl.ds(0, C1)], knb, sem).wait()

    def group_compute(qwb, knb, rbase, ebase):
        def edge_body(l, _):
            r = rbase + l
            kb = [knb[r, pl.ds(b * L, L)] for b in range(DB)]
            for h in range(H):
                acc = None
                for jb in range(PB):
                    q0, q1 = _unpack2(qwb[r, pl.ds(h * HW + jb * L, L)])
                    t = q0 * kb[2 * jb] + q1 * kb[2 * jb + 1]
                    acc = t if acc is None else acc + t
                stage_v[pl.ds((h * L + l) * L, L)] = acc
            return 0

        lax.fori_loop(0, L, edge_body, 0)
        ev = zero16
        for h in range(H):
            row = (h * L + lane) * L
            d = plsc.load_gather(stage_v, [row])
            for p in range(1, L):
                d = d + plsc.load_gather(stage_v, [row + p])
            ev = ev + jnp.exp(d)
        ev = ev * 0.25
        ev_v[pl.ds(ebase, L)] = ev
        dst16 = dst_v[pl.ds(ebase, L)]
        plsc.addupdate_scatter(z_v, [dst16], ev)

    def compute(i, qwb, knb):
        def g_body(g, _):
            group_compute(qwb, knb, g * L, i * C1 + g * L)
            return 0
        lax.fori_loop(0, G1, g_body, 0)

    fire(0, qw_a, kn_a, sem_a)

    def pair_body(t, _):
        i = 2 * t
        fire(i + 1, qw_b, kn_b, sem_b)
        waitg(qw_a, kn_a, sem_a)
        compute(i, qw_a, kn_a)

        @pl.when(t < PAIRS1 - 1)
        def _():
            fire(i + 2, qw_a, kn_a, sem_a)

        waitg(qw_b, kn_b, sem_b)
        compute(i + 1, qw_b, kn_b)
        return 0

    lax.fori_loop(0, PAIRS1, pair_body, 0)

    tq = pltpu.async_copy(qw_hbm.at[dst_v.at[pl.ds(TB1, T1)]],
                          qw_a.at[pl.ds(0, T1)], sem_a)
    tk = pltpu.async_copy(kn_hbm.at[src_v.at[pl.ds(TB1, T1)]],
                          kn_a.at[pl.ds(0, T1)], sem_a)
    tq.wait()
    tk.wait()
    group_compute(qw_a, kn_a, 0, TB1)

    pltpu.sync_copy(ev_v, expv_hbm.at[pl.ds(base0, EPW)])
    pltpu.sync_copy(z_v, zall_hbm.at[pl.ds(wid * N, N)])


def _sc_pass1(qw, kn, dst, src):
    mesh = plsc.VectorSubcoreMesh(core_axis_name="c", subcore_axis_name="s",
                                  num_cores=NC, num_subcores=NS)
    return pl.kernel(
        _pass1_body,
        compiler_params=pltpu.CompilerParams(needs_layout_passes=False),
        out_type=[
            jax.ShapeDtypeStruct((E,), jnp.float32),
            jax.ShapeDtypeStruct((NW * N,), jnp.float32),
        ],
        mesh=mesh,
        scratch_types=[
            pltpu.VMEM((EPW,), jnp.int32),
            pltpu.VMEM((EPW,), jnp.int32),
            pltpu.VMEM((C1, QWW), jnp.float32),
            pltpu.VMEM((C1, QWW), jnp.float32),
            pltpu.VMEM((C1, D), jnp.float32),
            pltpu.VMEM((C1, D), jnp.float32),
            pltpu.VMEM((EPW,), jnp.float32),
            pltpu.VMEM((N,), jnp.float32),
            pltpu.VMEM((H * L * L,), jnp.float32),
            pltpu.SemaphoreType.DMA,
            pltpu.SemaphoreType.DMA,
        ],
    )(qw, kn, dst, src)


def _mid_body(zall_ref, ev_ref, szinv_ref, sexpv_ref):
    z = jnp.sum(zall_ref[...], axis=0)
    z = jnp.where(z == 0.0, 1.0, z)
    szinv_ref[...] = lax.rsqrt(z)
    sexpv_ref[...] = jnp.sqrt(ev_ref[...])


def _tc_mid(zall, expv):
    za = zall.reshape(NW, 80, 125)
    ev = expv.reshape(2500, 128)
    szinv, sexpv = pl.pallas_call(
        _mid_body,
        out_shape=[
            jax.ShapeDtypeStruct((80, 125), jnp.float32),
            jax.ShapeDtypeStruct((2500, 128), jnp.float32),
        ],
    )(za, ev)
    return szinv.reshape(N), sexpv.reshape(E)


C2 = 80
NCH2 = EPW // C2
G2 = C2 // L


def _pass2_body(vn_hbm, dst_hbm, src_hbm, se_hbm, szi_hbm, fout_hbm,
                dc_a, dc_b, sc_a, sc_b, se_a, se_b, szg_a, szg_b,
                vr_a, vr_b, sr_a, sr_b, ds_a, ds_b, w_v, fout_sh,
                sem_ia, sem_ib, sem_ga, sem_gb, sem_sa, sem_sb):
    cid = lax.axis_index("c")
    sid = lax.axis_index("s")
    wid = sid * NC + cid
    base0 = wid * EPW
    zero16 = jnp.zeros((L,), jnp.float32)

    def _zr(i, _):
        for b in range(DB):
            sr_a[i, pl.ds(b * L, L)] = zero16
        return 0
    lax.fori_loop(0, C2, _zr, 0)

    def _zs(t, _):
        j = sid + t * NS

        @pl.when(j < NCH2)
        def _():
            pltpu.sync_copy(sr_a, fout_sh.at[pl.ds(pl.multiple_of(j * C2, 8), C2)])
        return 0
    lax.fori_loop(0, (NCH2 + NS - 1) // NS, _zs, 0)
    plsc.subcore_barrier()

    def fire_idx(j, dc, scb, seb, sem):
        pltpu.async_copy(dst_hbm.at[pl.ds(base0 + j * C2, C2)], dc, sem)
        pltpu.async_copy(src_hbm.at[pl.ds(base0 + j * C2, C2)], scb, sem)
        pltpu.async_copy(se_hbm.at[pl.ds(base0 + j * C2, C2)], seb, sem)

    def wait_idx(dc, scb, seb, sem):
        pltpu.make_async_copy(dst_hbm.at[pl.ds(0, C2)], dc, sem).wait()
        pltpu.make_async_copy(src_hbm.at[pl.ds(0, C2)], scb, sem).wait()
        pltpu.make_async_copy(se_hbm.at[pl.ds(0, C2)], seb, sem).wait()

    def fire_g(dc, scb, vrb, szgb, sem):
        pltpu.async_copy(vn_hbm.at[scb], vrb, sem)
        pltpu.async_copy(szi_hbm.at[dc], szgb, sem)

    def wait_g(vrb, szgb, sem):
        pltpu.make_async_copy(vn_hbm.at[pl.ds(0, C2)], vrb, sem).wait()
        pltpu.make_async_copy(se_hbm.at[pl.ds(0, C2)], szgb, sem).wait()

    def fire_s(srb, dsb, sem):
        pltpu.async_copy(srb, fout_sh.at[dsb], sem, add=True)

    def wait_s(srb, sem):
        pltpu.make_async_copy(vn_hbm.at[pl.ds(0, C2)], srb, sem).wait()

    def compute(dc, seb, szgb, vrb, srb, dsb):
        def g_body(g, _):
            off = g * L
            w_v[pl.ds(off, L)] = seb[pl.ds(off, L)] * szgb[pl.ds(off, L)]
            dsb[pl.ds(off, L)] = dc[pl.ds(off, L)]
            return 0
        lax.fori_loop(0, G2, g_body, 0)

        def row_body(r, _):
            wspl = plsc.load_gather(w_v, [jnp.full((L,), r, jnp.int32)])
            for b in range(DB):
                srb[r, pl.ds(b * L, L)] = vrb[r, pl.ds(b * L, L)] * wspl
            return 0
        lax.fori_loop(0, C2, row_body, 0)

    bufs = ((dc_a, sc_a, se_a, szg_a, vr_a, sr_a, ds_a, sem_ia, sem_ga, sem_sa),
            (dc_b, sc_b, se_b, szg_b, vr_b, sr_b, ds_b, sem_ib, sem_gb, sem_sb))

    def phase(j, t, par):
        dc, scb, seb, szgb, vrb, srb, dsb, sem_i, sem_g, sem_s = bufs[par]
        dcq, scq, seq, szgq, vrq, srq, dsq, sem_iq, sem_gq, sem_sq = bufs[1 - par]
        @pl.when(j + 1 < NCH2)
        def _():
            wait_idx(dcq, scq, seq, sem_iq)
            fire_g(dcq, scq, vrq, szgq, sem_gq)
        wait_g(vrb, szgb, sem_g)

        @pl.when(j >= 2)
        def _():
            wait_s(srb, sem_s)

        compute(dc, seb, szgb, vrb, srb, dsb)
        fire_s(srb, dsb, sem_s)

        @pl.when(j + 2 < NCH2)
        def _():
            fire_idx(j + 2, dc, scb, seb, sem_i)

    fire_idx(0, dc_a, sc_a, se_a, sem_ia)
    fire_idx(1, dc_b, sc_b, se_b, sem_ib)
    wait_idx(dc_a, sc_a, se_a, sem_ia)
    fire_g(dc_a, sc_a, vr_a, szg_a, sem_ga)

    def pair_body(t, _):
        j = 2 * t
        phase(j, t, 0)
        phase(j + 1, t, 1)
        return 0
    lax.fori_loop(0, NCH2 // 2, pair_body, 0)
    phase(NCH2 - 1, NCH2 // 2, 0)

    wait_s(sr_a, sem_sa)
    wait_s(sr_b, sem_sb)
    plsc.subcore_barrier()

    def _wb(t, _):
        j = sid + t * NS

        @pl.when(j < NCH2)
        def _():
            row0 = pl.multiple_of(j * C2, 8)
            pltpu.sync_copy(fout_sh.at[pl.ds(row0, C2)],
                            fout_hbm.at[cid, pl.ds(row0, C2)])
        return 0
    lax.fori_loop(0, (NCH2 + NS - 1) // NS, _wb, 0)


def _sc_pass2(vn, dst, src, sexpv, szinv):
    mesh = plsc.VectorSubcoreMesh(core_axis_name="c", subcore_axis_name="s",
                                  num_cores=NC, num_subcores=NS)
    return pl.kernel(
        _pass2_body,
        compiler_params=pltpu.CompilerParams(needs_layout_passes=False),
        out_type=jax.ShapeDtypeStruct((NC, N, D), jnp.float32),
        mesh=mesh,
        scratch_types=[
            pltpu.VMEM((C2,), jnp.int32),
            pltpu.VMEM((C2,), jnp.int32),
            pltpu.VMEM((C2,), jnp.int32),
            pltpu.VMEM((C2,), jnp.int32),
            pltpu.VMEM((C2,), jnp.float32),
            pltpu.VMEM((C2,), jnp.float32),
            pltpu.VMEM((C2,), jnp.float32),
            pltpu.VMEM((C2,), jnp.float32),
            pltpu.VMEM((C2, D), jnp.float32),
            pltpu.VMEM((C2, D), jnp.float32),
            pltpu.VMEM((C2, D), jnp.float32),
            pltpu.VMEM((C2, D), jnp.float32),
            pltpu.VMEM((C2,), jnp.int32),
            pltpu.VMEM((C2,), jnp.int32),
            pltpu.VMEM((C2,), jnp.float32),
            pltpu.VMEM_SHARED((N, D), jnp.float32),
            pltpu.SemaphoreType.DMA,
            pltpu.SemaphoreType.DMA,
            pltpu.SemaphoreType.DMA,
            pltpu.SemaphoreType.DMA,
            pltpu.SemaphoreType.DMA,
            pltpu.SemaphoreType.DMA,
        ],
    )(vn, dst, src, sexpv, szinv)


def _post_body(fp_ref, vn_ref, out_ref):
    out_ref[...] = fp_ref[0] + fp_ref[1] + vn_ref[...]


def _tc_post(fout, vn):
    rb = 1000
    return pl.pallas_call(
        _post_body,
        grid=(N // rb,),
        in_specs=[
            pl.BlockSpec((NC, rb, D), lambda i: (0, i, 0)),
            pl.BlockSpec((rb, D), lambda i: (i, 0)),
        ],
        out_specs=pl.BlockSpec((rb, D), lambda i: (i, 0)),
        out_shape=jax.ShapeDtypeStruct((N, D), jnp.float32),
    )(fout, vn)


@jax.jit
def kernel(node_features, edge_dst, edge_src, W_q, W_k, W_v, W_dot):
    dst = edge_dst.astype(jnp.int32)
    src = edge_src.astype(jnp.int32)
    wd2 = jnp.transpose(W_dot, (1, 0, 2)).reshape(D, H * D)

    qw, kn, vn = _tc_pre(node_features, W_q, W_k, W_v, wd2)
    expv, zall = _sc_pass1(_pack_pairs(qw), kn, dst, src)
    szinv, sexpv = _tc_mid(zall, expv)
    fout = _sc_pass2(vn, dst, src, sexpv, szinv)
    return _tc_post(fout, vn)
